# baseline (speedup 1.0000x reference)
import jax
import jax.numpy as jnp
from jax import lax
from jax.experimental import pallas as pl
from jax.experimental.pallas import tpu as pltpu
from jax.experimental.pallas import tpu_sc as plsc

N = 10000
D = 256
E = 160000

NTILE = 16
TPT = 10240
K = 80
C = TPT // K
KD = 128
CD = TPT // KD
ROWS_PT = 640
NP = 10240
NI = C // 4

def _mesh():
    return plsc.VectorSubcoreMesh(core_axis_name="c", subcore_axis_name="s")


def _deg_body(idx_hbm, ones_hbm, zeros_hbm, out_hbm, acc, idx_v0, idx_v1,
              ones_v, sem):
    c = lax.axis_index("c")
    s = lax.axis_index("s")
    pltpu.sync_copy(ones_hbm, ones_v)
    pltpu.sync_copy(zeros_hbm, acc.at[pl.ds(s * 2 * ROWS_PT, 2 * ROWS_PT)])
    plsc.subcore_barrier()
    for a, idx_v in ((0, idx_v0), (1, idx_v1)):
        pltpu.sync_copy(idx_hbm.at[(2 * c + a) * NTILE + s], idx_v)

    def chunk(j, carry):
        for idx_v in (idx_v0, idx_v1):
            pltpu.async_copy(ones_v, acc.at[idx_v.at[j]], sem, add=True)

        @pl.when(j >= 8)
        def _():
            for idx_v in (idx_v0, idx_v1):
                pltpu.make_async_copy(ones_v, acc.at[idx_v.at[j - 8]],
                                      sem).wait()
        return carry

    lax.fori_loop(0, CD, chunk, 0, unroll=False)

    def draintail(j, carry):
        for idx_v in (idx_v0, idx_v1):
            pltpu.make_async_copy(ones_v, acc.at[idx_v.at[j]], sem).wait()
        return carry

    lax.fori_loop(CD - 8, CD, draintail, 0, unroll=False)
    plsc.subcore_barrier()
    off = c * 2 * NP + s * 2 * ROWS_PT
    pltpu.sync_copy(acc.at[pl.ds(s * 2 * ROWS_PT, 2 * ROWS_PT)],
                    out_hbm.at[pl.ds(off, 2 * ROWS_PT)])


def _k_deg(idx4, ones, zeros):
    return pl.kernel(
        _deg_body,
        out_type=jax.ShapeDtypeStruct((4 * NP,), jnp.float32),
        mesh=_mesh(),
        scratch_types=[
            pltpu.VMEM_SHARED((2 * NP,), jnp.float32),
            pltpu.VMEM((CD, KD), jnp.int32),
            pltpu.VMEM((CD, KD), jnp.int32),
            pltpu.VMEM((KD,), jnp.float32),
            pltpu.SemaphoreType.DMA,
        ],
    )(idx4, ones, zeros)


def _agg_body(h_hbm, srcg_hbm, dstg_hbm, out_hbm, acc,
              si0, si1, si2, si3, di0, di1, di2, di3, g0, g1, g2, g3,
              gs0, gs1, gs2, gs3, is0, is1, is2, is3,
              ds0, ds1, ds2, ds3, ss0, ss1, ss2, ss3):
    c = lax.axis_index("c")
    s = lax.axis_index("s")
    si = (si0, si1, si2, si3)
    isem = (is0, is1, is2, is3)
    dbuf = (di0, di1, di2, di3)
    dsem = (ds0, ds1, ds2, ds3)
    gbuf = (g0, g1, g2, g3)
    gsem = (gs0, gs1, gs2, gs3)
    ssem = (ss0, ss1, ss2, ss3)
    srcg_row = srcg_hbm.at[c * NTILE + s]
    dstg_row = dstg_hbm.at[s]

    zv = jnp.zeros((16,), jnp.float32)

    def zrow(r, carry):
        for rr in range(2):
            for t in range(8):
                gbuf[0][2 * r + rr, pl.ds(16 * t, 16)] = zv
        return carry

    lax.fori_loop(0, K // 2, zrow, 0, unroll=False)
    for z in range(ROWS_PT // K):
        pltpu.sync_copy(gbuf[0], acc.at[pl.ds(s * ROWS_PT + z * K, K)])
    plsc.subcore_barrier()

    for m in range(4):
        pltpu.async_copy(srcg_row.at[pl.ds(m, 1)], si[m], isem[m])
    for m in range(3):
        pltpu.async_copy(dstg_row.at[pl.ds(m, 1)], dbuf[m], dsem[m])
    for m in range(3):
        pltpu.make_async_copy(srcg_row.at[pl.ds(m, 1)], si[m],
                              isem[m]).wait()
        pltpu.async_copy(h_hbm.at[si[m].at[0]], gbuf[m], gsem[m])

    def step(i, carry):
        for u in range(4):
            j = i * 4 + u
            u3 = (u + 3) % 4
            pltpu.make_async_copy(h_hbm.at[si[u].at[0]], gbuf[u],
                                  gsem[u]).wait()

            @pl.when(i < NI - 1)
            def _():
                pltpu.async_copy(srcg_row.at[pl.ds(j + 4, 1)], si[u], isem[u])

            def _drain():
                pltpu.make_async_copy(gbuf[u3], acc.at[dbuf[u3].at[0]],
                                      ssem[u3]).wait()

            if u > 0:
                _drain()
            else:
                pl.when(i > 0)(_drain)

            def _ahead():
                pltpu.async_copy(dstg_row.at[pl.ds(j + 3, 1)], dbuf[u3],
                                 dsem[u3])
                pltpu.make_async_copy(srcg_row.at[pl.ds(j + 3, 1)], si[u3],
                                      isem[u3]).wait()
                pltpu.async_copy(h_hbm.at[si[u3].at[0]], gbuf[u3], gsem[u3])

            if u == 0:
                _ahead()
            else:
                pl.when(i < NI - 1)(_ahead)

            pltpu.make_async_copy(dstg_row.at[pl.ds(j, 1)], dbuf[u],
                                  dsem[u]).wait()
            pltpu.async_copy(gbuf[u], acc.at[dbuf[u].at[0]], ssem[u],
                             add=True)
        return carry

    lax.fori_loop(0, NI, step, 0, unroll=False)
    pltpu.make_async_copy(gbuf[3], acc.at[dbuf[3].at[0]], ssem[3]).wait()
    plsc.subcore_barrier()
    pltpu.sync_copy(acc.at[pl.ds(s * ROWS_PT, ROWS_PT)],
                    out_hbm.at[pl.ds(c * NP + s * ROWS_PT, ROWS_PT)])


def _k_agg(h_flat, srcg, dstg):
    return pl.kernel(
        _agg_body,
        out_type=jax.ShapeDtypeStruct((2 * NP, 128), jnp.float32),
        mesh=_mesh(),
        scratch_types=[
            pltpu.VMEM_SHARED((NP, 128), jnp.float32),
            pltpu.VMEM((1, K), jnp.int32),
            pltpu.VMEM((1, K), jnp.int32),
            pltpu.VMEM((1, K), jnp.int32),
            pltpu.VMEM((1, K), jnp.int32),
            pltpu.VMEM((1, K), jnp.int32),
            pltpu.VMEM((1, K), jnp.int32),
            pltpu.VMEM((1, K), jnp.int32),
            pltpu.VMEM((1, K), jnp.int32),
            pltpu.VMEM((K, 128), jnp.float32),
            pltpu.VMEM((K, 128), jnp.float32),
            pltpu.VMEM((K, 128), jnp.float32),
            pltpu.VMEM((K, 128), jnp.float32),
        ] + [pltpu.SemaphoreType.DMA] * 16,
    )(h_flat, srcg, dstg)


BN = 1000


def _mm1_body(x_ref, cnt_ref, w_ref, out_ref):
    s = lax.rsqrt(jnp.maximum(cnt_ref[...], 1.0))
    out_ref[...] = jnp.dot(x_ref[...] * s, w_ref[...],
                           preferred_element_type=jnp.float32)


def _k_mm1(x, cnt, w):
    nb = N // BN
    return pl.pallas_call(
        _mm1_body,
        grid=(nb, 2),
        in_specs=[
            pl.BlockSpec((BN, D), lambda i, h: (i, 0)),
            pl.BlockSpec((BN, 1), lambda i, h: (i, 0)),
            pl.BlockSpec((D, 128), lambda i, h: (0, h)),
        ],
        out_specs=pl.BlockSpec((BN, 128), lambda i, h: (h * nb + i, 0)),
        out_shape=jax.ShapeDtypeStruct((2 * N, 128), jnp.float32),
    )(x, cnt, w)


def _mm2_body(a_ref, cin_ref, cout_ref, b_ref, w_ref, out_ref):
    x = jnp.concatenate([a_ref[0], a_ref[1]], axis=1)
    si = lax.rsqrt(jnp.maximum(cin_ref[...], 1.0))
    so = lax.rsqrt(jnp.maximum(cout_ref[...], 1.0))
    h = jnp.maximum(x * si + b_ref[...], 0.0) * so
    out_ref[...] = jnp.dot(h, w_ref[...], preferred_element_type=jnp.float32)


def _k_mm2(agg, cin, cout, b1, w):
    nb = N // BN
    return pl.pallas_call(
        _mm2_body,
        grid=(nb, 2),
        in_specs=[
            pl.BlockSpec((2, BN, 128), lambda i, h: (0, i, 0)),
            pl.BlockSpec((BN, 1), lambda i, h: (i, 0)),
            pl.BlockSpec((BN, 1), lambda i, h: (i, 0)),
            pl.BlockSpec((1, D), lambda i, h: (0, 0)),
            pl.BlockSpec((D, 128), lambda i, h: (0, h)),
        ],
        out_specs=pl.BlockSpec((BN, 128), lambda i, h: (h * nb + i, 0)),
        out_shape=jax.ShapeDtypeStruct((2 * N, 128), jnp.float32),
    )(agg, cin, cout, b1, w)


def _fin_body(a_ref, cin_ref, b_ref, out_ref, acc_ref):
    p = pl.program_id(0)
    i = pl.program_id(1)
    si = lax.rsqrt(jnp.maximum(cin_ref[...], 1.0))
    y = jnp.concatenate([a_ref[0], a_ref[1]], axis=1) * si + b_ref[...]

    @pl.when(jnp.logical_and(p == 0, i == 0))
    def _():
        acc_ref[...] = jnp.zeros_like(acc_ref)

    @pl.when(p == 0)
    def _():
        acc_ref[0, :] += jnp.sum(y, axis=0)
        acc_ref[1, :] += jnp.sum(y * y, axis=0)

    @pl.when(p == 1)
    def _():
        n = jnp.float32(N)
        mean = acc_ref[0, :] / n
        var = (acc_ref[1, :] - n * mean * mean) / (n - 1.0)
        out_ref[...] = (y - mean[None, :]) * lax.rsqrt(var)[None, :]


def _k_fin(agg, cin, b2):
    nb = N // BN
    return pl.pallas_call(
        _fin_body,
        grid=(2, nb),
        in_specs=[
            pl.BlockSpec((2, BN, 128), lambda p, i: (0, i, 0)),
            pl.BlockSpec((BN, 1), lambda p, i: (i, 0)),
            pl.BlockSpec((1, D), lambda p, i: (0, 0)),
        ],
        out_specs=pl.BlockSpec((BN, D), lambda p, i: (i, 0)),
        out_shape=jax.ShapeDtypeStruct((N, D), jnp.float32),
        scratch_shapes=[pltpu.VMEM((2, D), jnp.float32)],
    )(agg, cin, b2)


def _tile_pad(idx):
    per = E // NTILE
    t = idx.reshape(NTILE, per)
    pad = TPT - per
    scrap = (N + (jnp.arange(pad, dtype=jnp.int32) % (NP - N)))
    return jnp.concatenate(
        [t, jnp.broadcast_to(scrap, (NTILE, pad))], axis=1)


def _gather_pad(idx):
    per = E // NTILE
    t = idx.reshape(NTILE, per)
    pad = TPT - per
    fill = (jnp.arange(pad, dtype=jnp.int32) * 37) % N
    return jnp.concatenate(
        [t, jnp.broadcast_to(fill, (NTILE, pad))], axis=1)


def kernel(graph1, feat1, graph2, feat2, W1, b1, W2, b2):
    f32 = jnp.float32
    ones_deg = jnp.ones((KD,), f32)
    zeros_deg = jnp.zeros((2 * ROWS_PT,), f32)
    b1r = b1.reshape(1, D)
    b2r = b2.reshape(1, D)

    idx4 = jnp.stack([
        _tile_pad(graph1[0]), _tile_pad(graph1[1]) + NP,
        _tile_pad(graph2[0]), _tile_pad(graph2[1]) + NP,
    ]).reshape(4 * NTILE, CD, KD)
    cnts = _k_deg(idx4, ones_deg, zeros_deg).reshape(4, NP)
    co1 = cnts[0, :N].reshape(N, 1)
    ci1 = cnts[1, :N].reshape(N, 1)
    co2 = cnts[2, :N].reshape(N, 1)
    ci2 = cnts[3, :N].reshape(N, 1)

    def prep(graph):
        src_t = _gather_pad(graph[0])
        dst_t = _tile_pad(graph[1])
        srcg = jnp.stack([src_t, src_t + N]).reshape(2 * NTILE, C, K)
        dstg = dst_t.reshape(NTILE, C, K)
        return srcg, dstg

    srcg1, dstg1 = prep(graph1)
    srcg2, dstg2 = prep(graph2)

    h1 = _k_mm1(feat1, co1, W1)
    h2 = _k_mm1(feat2, co2, W1)
    a11 = _k_agg(h1, srcg1, dstg1).reshape(2, NP, 128)
    x1 = _k_mm2(a11, ci1, co1, b1r, W2)
    a21 = _k_agg(h2, srcg2, dstg2).reshape(2, NP, 128)
    a12 = _k_agg(x1, srcg1, dstg1).reshape(2, NP, 128)
    x2 = _k_mm2(a21, ci2, co2, b1r, W2)
    a22 = _k_agg(x2, srcg2, dstg2).reshape(2, NP, 128)
    z1 = _k_fin(a12, ci1, b2r)
    z2 = _k_fin(a22, ci2, b2r)
    return (z1, z2)


# speedup vs baseline: 1.0164x; 1.0164x over previous
import jax
import jax.numpy as jnp
from jax import lax
from jax.experimental import pallas as pl
from jax.experimental.pallas import tpu as pltpu
from jax.experimental.pallas import tpu_sc as plsc

N = 10000
D = 256
E = 160000

NTILE = 16
TPT = 10240
K = 80
C = TPT // K
KD = 128
CD = TPT // KD
ROWS_PT = 640
NP = 10240
NI = C // 4

def _mesh():
    return plsc.VectorSubcoreMesh(core_axis_name="c", subcore_axis_name="s")


def _deg_body(idx_hbm, ones_hbm, zeros_hbm, out_hbm, acc, idx_v0, idx_v1,
              ones_v, sem):
    c = lax.axis_index("c")
    s = lax.axis_index("s")
    pltpu.sync_copy(ones_hbm, ones_v)
    pltpu.sync_copy(zeros_hbm, acc.at[pl.ds(s * 2 * ROWS_PT, 2 * ROWS_PT)])
    plsc.subcore_barrier()
    for a, idx_v in ((0, idx_v0), (1, idx_v1)):
        pltpu.sync_copy(idx_hbm.at[(2 * c + a) * NTILE + s], idx_v)

    def chunk(j, carry):
        for idx_v in (idx_v0, idx_v1):
            pltpu.async_copy(ones_v, acc.at[idx_v.at[j]], sem, add=True)

        @pl.when(j >= 8)
        def _():
            for idx_v in (idx_v0, idx_v1):
                pltpu.make_async_copy(ones_v, acc.at[idx_v.at[j - 8]],
                                      sem).wait()
        return carry

    lax.fori_loop(0, CD, chunk, 0, unroll=False)

    def draintail(j, carry):
        for idx_v in (idx_v0, idx_v1):
            pltpu.make_async_copy(ones_v, acc.at[idx_v.at[j]], sem).wait()
        return carry

    lax.fori_loop(CD - 8, CD, draintail, 0, unroll=False)
    plsc.subcore_barrier()
    off = c * 2 * NP + s * 2 * ROWS_PT
    pltpu.sync_copy(acc.at[pl.ds(s * 2 * ROWS_PT, 2 * ROWS_PT)],
                    out_hbm.at[pl.ds(off, 2 * ROWS_PT)])


def _k_deg(idx4, ones, zeros):
    return pl.kernel(
        _deg_body,
        out_type=jax.ShapeDtypeStruct((4 * NP,), jnp.float32),
        mesh=_mesh(),
        scratch_types=[
            pltpu.VMEM_SHARED((2 * NP,), jnp.float32),
            pltpu.VMEM((CD, KD), jnp.int32),
            pltpu.VMEM((CD, KD), jnp.int32),
            pltpu.VMEM((KD,), jnp.float32),
            pltpu.SemaphoreType.DMA,
        ],
    )(idx4, ones, zeros)


def _agg_body(h_hbm, srcg_hbm, dstg_hbm, out_hbm, acc,
              si0, si1, si2, si3, di0, di1, di2, di3, g0, g1, g2, g3,
              gs0, gs1, gs2, gs3, is0, is1, is2, is3,
              ds0, ds1, ds2, ds3, ss0, ss1, ss2, ss3):
    c = lax.axis_index("c")
    s = lax.axis_index("s")
    si = (si0, si1, si2, si3)
    isem = (is0, is1, is2, is3)
    dbuf = (di0, di1, di2, di3)
    dsem = (ds0, ds1, ds2, ds3)
    gbuf = (g0, g1, g2, g3)
    gsem = (gs0, gs1, gs2, gs3)
    ssem = (ss0, ss1, ss2, ss3)
    srcg_row = srcg_hbm.at[c * NTILE + s]
    dstg_row = dstg_hbm.at[s]

    zv = jnp.zeros((16,), jnp.float32)

    def zrow(r, carry):
        for rr in range(2):
            for t in range(8):
                gbuf[0][2 * r + rr, pl.ds(16 * t, 16)] = zv
        return carry

    lax.fori_loop(0, K // 2, zrow, 0, unroll=False)
    for z in range(ROWS_PT // K):
        pltpu.sync_copy(gbuf[0], acc.at[pl.ds(s * ROWS_PT + z * K, K)])
    plsc.subcore_barrier()

    for m in range(4):
        pltpu.async_copy(srcg_row.at[pl.ds(m, 1)], si[m], isem[m])
    for m in range(3):
        pltpu.async_copy(dstg_row.at[pl.ds(m, 1)], dbuf[m], dsem[m])
    for m in range(3):
        pltpu.make_async_copy(srcg_row.at[pl.ds(m, 1)], si[m],
                              isem[m]).wait()
        pltpu.async_copy(h_hbm.at[si[m].at[0]], gbuf[m], gsem[m])

    def step(i, carry):
        for u in range(4):
            j = i * 4 + u
            u3 = (u + 3) % 4
            pltpu.make_async_copy(h_hbm.at[si[u].at[0]], gbuf[u],
                                  gsem[u]).wait()

            @pl.when(i < NI - 1)
            def _():
                pltpu.async_copy(srcg_row.at[pl.ds(j + 4, 1)], si[u], isem[u])

            def _drain():
                pltpu.make_async_copy(gbuf[u3], acc.at[dbuf[u3].at[0]],
                                      ssem[u3]).wait()

            if u > 0:
                _drain()
            else:
                pl.when(i > 0)(_drain)

            def _ahead():
                pltpu.async_copy(dstg_row.at[pl.ds(j + 3, 1)], dbuf[u3],
                                 dsem[u3])
                pltpu.make_async_copy(srcg_row.at[pl.ds(j + 3, 1)], si[u3],
                                      isem[u3]).wait()
                pltpu.async_copy(h_hbm.at[si[u3].at[0]], gbuf[u3], gsem[u3])

            if u == 0:
                _ahead()
            else:
                pl.when(i < NI - 1)(_ahead)

            pltpu.make_async_copy(dstg_row.at[pl.ds(j, 1)], dbuf[u],
                                  dsem[u]).wait()
            pltpu.async_copy(gbuf[u], acc.at[dbuf[u].at[0]], ssem[u],
                             add=True)
        return carry

    lax.fori_loop(0, NI, step, 0, unroll=False)
    pltpu.make_async_copy(gbuf[3], acc.at[dbuf[3].at[0]], ssem[3]).wait()
    plsc.subcore_barrier()
    pltpu.sync_copy(acc.at[pl.ds(s * ROWS_PT, ROWS_PT)],
                    out_hbm.at[pl.ds(c * NP + s * ROWS_PT, ROWS_PT)])


def _k_agg(h_flat, srcg, dstg):
    return pl.kernel(
        _agg_body,
        out_type=jax.ShapeDtypeStruct((2 * NP, 128), jnp.float32),
        mesh=_mesh(),
        scratch_types=[
            pltpu.VMEM_SHARED((NP, 128), jnp.float32),
            pltpu.VMEM((1, K), jnp.int32),
            pltpu.VMEM((1, K), jnp.int32),
            pltpu.VMEM((1, K), jnp.int32),
            pltpu.VMEM((1, K), jnp.int32),
            pltpu.VMEM((1, K), jnp.int32),
            pltpu.VMEM((1, K), jnp.int32),
            pltpu.VMEM((1, K), jnp.int32),
            pltpu.VMEM((1, K), jnp.int32),
            pltpu.VMEM((K, 128), jnp.float32),
            pltpu.VMEM((K, 128), jnp.float32),
            pltpu.VMEM((K, 128), jnp.float32),
            pltpu.VMEM((K, 128), jnp.float32),
        ] + [pltpu.SemaphoreType.DMA] * 16,
    )(h_flat, srcg, dstg)


BN = 2000


def _mm1_body(x_ref, cnt_ref, w_ref, out_ref):
    s = lax.rsqrt(jnp.maximum(cnt_ref[...], 1.0))
    out_ref[...] = jnp.dot(x_ref[...] * s, w_ref[...],
                           preferred_element_type=jnp.float32)


def _k_mm1(x, cnt, w):
    nb = N // BN
    return pl.pallas_call(
        _mm1_body,
        grid=(nb, 2),
        in_specs=[
            pl.BlockSpec((BN, D), lambda i, h: (i, 0)),
            pl.BlockSpec((BN, 1), lambda i, h: (i, 0)),
            pl.BlockSpec((D, 128), lambda i, h: (0, h)),
        ],
        out_specs=pl.BlockSpec((BN, 128), lambda i, h: (h * nb + i, 0)),
        out_shape=jax.ShapeDtypeStruct((2 * N, 128), jnp.float32),
    )(x, cnt, w)


def _mm2_body(a_ref, cin_ref, cout_ref, b_ref, w_ref, out_ref):
    x = jnp.concatenate([a_ref[0], a_ref[1]], axis=1)
    si = lax.rsqrt(jnp.maximum(cin_ref[...], 1.0))
    so = lax.rsqrt(jnp.maximum(cout_ref[...], 1.0))
    h = jnp.maximum(x * si + b_ref[...], 0.0) * so
    out_ref[...] = jnp.dot(h, w_ref[...], preferred_element_type=jnp.float32)


def _k_mm2(agg, cin, cout, b1, w):
    nb = N // BN
    return pl.pallas_call(
        _mm2_body,
        grid=(nb, 2),
        in_specs=[
            pl.BlockSpec((2, BN, 128), lambda i, h: (0, i, 0)),
            pl.BlockSpec((BN, 1), lambda i, h: (i, 0)),
            pl.BlockSpec((BN, 1), lambda i, h: (i, 0)),
            pl.BlockSpec((1, D), lambda i, h: (0, 0)),
            pl.BlockSpec((D, 128), lambda i, h: (0, h)),
        ],
        out_specs=pl.BlockSpec((BN, 128), lambda i, h: (h * nb + i, 0)),
        out_shape=jax.ShapeDtypeStruct((2 * N, 128), jnp.float32),
    )(agg, cin, cout, b1, w)


def _fin_body(a_ref, cin_ref, b_ref, out_ref, acc_ref):
    p = pl.program_id(0)
    i = pl.program_id(1)
    si = lax.rsqrt(jnp.maximum(cin_ref[...], 1.0))
    y = jnp.concatenate([a_ref[0], a_ref[1]], axis=1) * si + b_ref[...]

    @pl.when(jnp.logical_and(p == 0, i == 0))
    def _():
        acc_ref[...] = jnp.zeros_like(acc_ref)

    @pl.when(p == 0)
    def _():
        acc_ref[0, :] += jnp.sum(y, axis=0)
        acc_ref[1, :] += jnp.sum(y * y, axis=0)

    @pl.when(p == 1)
    def _():
        n = jnp.float32(N)
        mean = acc_ref[0, :] / n
        var = (acc_ref[1, :] - n * mean * mean) / (n - 1.0)
        out_ref[...] = (y - mean[None, :]) * lax.rsqrt(var)[None, :]


def _k_fin(agg, cin, b2):
    nb = N // BN
    return pl.pallas_call(
        _fin_body,
        grid=(2, nb),
        in_specs=[
            pl.BlockSpec((2, BN, 128), lambda p, i: (0, i, 0)),
            pl.BlockSpec((BN, 1), lambda p, i: (i, 0)),
            pl.BlockSpec((1, D), lambda p, i: (0, 0)),
        ],
        out_specs=pl.BlockSpec((BN, D), lambda p, i: (i, 0)),
        out_shape=jax.ShapeDtypeStruct((N, D), jnp.float32),
        scratch_shapes=[pltpu.VMEM((2, D), jnp.float32)],
    )(agg, cin, b2)


def _tile_pad(idx):
    per = E // NTILE
    t = idx.reshape(NTILE, per)
    pad = TPT - per
    scrap = (N + (jnp.arange(pad, dtype=jnp.int32) % (NP - N)))
    return jnp.concatenate(
        [t, jnp.broadcast_to(scrap, (NTILE, pad))], axis=1)


def _gather_pad(idx):
    per = E // NTILE
    t = idx.reshape(NTILE, per)
    pad = TPT - per
    fill = (jnp.arange(pad, dtype=jnp.int32) * 37) % N
    return jnp.concatenate(
        [t, jnp.broadcast_to(fill, (NTILE, pad))], axis=1)


def kernel(graph1, feat1, graph2, feat2, W1, b1, W2, b2):
    f32 = jnp.float32
    ones_deg = jnp.ones((KD,), f32)
    zeros_deg = jnp.zeros((2 * ROWS_PT,), f32)
    b1r = b1.reshape(1, D)
    b2r = b2.reshape(1, D)

    idx4 = jnp.stack([
        _tile_pad(graph1[0]), _tile_pad(graph1[1]) + NP,
        _tile_pad(graph2[0]), _tile_pad(graph2[1]) + NP,
    ]).reshape(4 * NTILE, CD, KD)
    cnts = _k_deg(idx4, ones_deg, zeros_deg).reshape(4, NP)
    co1 = cnts[0, :N].reshape(N, 1)
    ci1 = cnts[1, :N].reshape(N, 1)
    co2 = cnts[2, :N].reshape(N, 1)
    ci2 = cnts[3, :N].reshape(N, 1)

    def prep(graph):
        src_t = _gather_pad(graph[0])
        dst_t = _tile_pad(graph[1])
        srcg = jnp.stack([src_t, src_t + N]).reshape(2 * NTILE, C, K)
        dstg = dst_t.reshape(NTILE, C, K)
        return srcg, dstg

    srcg1, dstg1 = prep(graph1)
    srcg2, dstg2 = prep(graph2)

    h1 = _k_mm1(feat1, co1, W1)
    h2 = _k_mm1(feat2, co2, W1)
    a11 = _k_agg(h1, srcg1, dstg1).reshape(2, NP, 128)
    x1 = _k_mm2(a11, ci1, co1, b1r, W2)
    a21 = _k_agg(h2, srcg2, dstg2).reshape(2, NP, 128)
    a12 = _k_agg(x1, srcg1, dstg1).reshape(2, NP, 128)
    x2 = _k_mm2(a21, ci2, co2, b1r, W2)
    a22 = _k_agg(x2, srcg2, dstg2).reshape(2, NP, 128)
    z1 = _k_fin(a12, ci1, b2r)
    z2 = _k_fin(a22, ci2, b2r)
    return (z1, z2)
